# baseline (speedup 1.0000x reference)
import functools

import jax
import jax.numpy as jnp
from jax.experimental import pallas as pl
from jax.experimental.pallas import tpu as pltpu

B, N = 2, 100000
H, WG = 256, 256
FD = 128
EPS = 1e-5
TOT = B * N
BLK = 8000
NBLK = TOT // BLK


def _mlp_body(pts_ref, a1_ref, c1_ref, a2_ref, c2_ref, a3_ref, c3_ref,
              feat_ref, idx_ref):
    x = pts_ref[...]
    h = jnp.maximum(jnp.dot(x, a1_ref[...], preferred_element_type=jnp.float32)
                    + c1_ref[...], 0.0)
    h = jnp.maximum(jnp.dot(h, a2_ref[...], preferred_element_type=jnp.float32)
                    + c2_ref[...], 0.0)
    h = jnp.maximum(jnp.dot(h, a3_ref[...], preferred_element_type=jnp.float32)
                    + c3_ref[...], 0.0)
    feat_ref[...] = h

    bid = pl.program_id(0)
    gstart = bid * BLK
    row = gstart + jax.lax.broadcasted_iota(jnp.int32, (BLK, 1), 0)
    b = row // N
    xx = x[:, 0:1]
    yy = x[:, 1:2]
    xn = (xx + 50.0) * 0.01
    yn = (yy + 50.0) * 0.01
    valid = (xn >= 0) & (xn <= 1) & (yn >= 0) & (yn <= 1)
    gx = jnp.clip((xn * (WG - 1)).astype(jnp.int32), 0, WG - 1)
    gy = jnp.clip((yn * (H - 1)).astype(jnp.int32), 0, H - 1)
    flat = b * (H * WG) + gy * WG + gx
    flat = jnp.where(valid, flat, B * H * WG)
    idx_ref[...] = flat


@jax.jit
def _mlp(pts, a1, c1, a2, c2, a3, c3):
    return pl.pallas_call(
        _mlp_body,
        grid=(NBLK,),
        in_specs=[
            pl.BlockSpec((BLK, 4), lambda i: (i, 0)),
            pl.BlockSpec((4, 64), lambda i: (0, 0)),
            pl.BlockSpec((1, 64), lambda i: (0, 0)),
            pl.BlockSpec((64, 128), lambda i: (0, 0)),
            pl.BlockSpec((1, 128), lambda i: (0, 0)),
            pl.BlockSpec((128, 128), lambda i: (0, 0)),
            pl.BlockSpec((1, 128), lambda i: (0, 0)),
        ],
        out_specs=[
            pl.BlockSpec((BLK, 128), lambda i: (i, 0)),
            pl.BlockSpec((BLK, 1), lambda i: (i, 0)),
        ],
        out_shape=[
            jax.ShapeDtypeStruct((TOT, 128), jnp.float32),
            jax.ShapeDtypeStruct((TOT, 1), jnp.int32),
        ],
    )(pts, a1, c1, a2, c2, a3, c3)


def kernel(points, W1, b1, g1, be1, W2, b2, g2, be2, W3, b3, g3, be3):
    s = 1.0 / jnp.sqrt(1.0 + EPS)
    a1 = (W1.T * (g1 * s)).astype(jnp.float32)
    c1 = (b1 * g1 * s + be1)[None, :]
    a2 = (W2.T * (g2 * s)).astype(jnp.float32)
    c2 = (b2 * g2 * s + be2)[None, :]
    a3 = (W3.T * (g3 * s)).astype(jnp.float32)
    c3 = (b3 * g3 * s + be3)[None, :]
    pts = points.reshape(TOT, 4)
    feat, idx = _mlp(pts, a1, c1, a2, c2, a3, c3)
    flat = idx.reshape(-1)
    fm = jnp.zeros((B * H * WG + 1, FD), dtype=points.dtype)
    fm = fm.at[flat].max(feat)
    fm = fm[: B * H * WG]
    return fm.reshape(B, H, WG, FD).transpose(0, 3, 1, 2)
